# baseline (speedup 1.0000x reference)
import functools

import jax
import jax.numpy as jnp
from jax import lax
from jax.experimental import pallas as pl
from jax.experimental.pallas import tpu as pltpu
from jax.experimental.pallas import tpu_sc as plsc

_SEQ = 200
_D = 64
_LANES = 16
_SCALE = 8.0

_info = plsc.get_sparse_core_info()
_NC = _info.num_cores
_NS = _info.num_subcores
_NW = _NC * _NS


def _build_sc_gather(BATCH: int, V: int):
    ST = _SEQ // 8
    BT = BATCH // 128
    DT = _D // 8
    assert BT == _NW

    mesh = plsc.VectorSubcoreMesh(core_axis_name="c", subcore_axis_name="s")

    @functools.partial(
        pl.kernel,
        out_type=jax.ShapeDtypeStruct((_SEQ, DT, BT, 8, 128), jnp.float32),
        mesh=mesh,
        scratch_types=[
            pltpu.VMEM((512,), jnp.int32),
            pltpu.VMEM((512,), jnp.int32),
            pltpu.VMEM((_SEQ, _D), jnp.float32),
            pltpu.VMEM((512, _D), jnp.float32),
            pltpu.VMEM((512, _D), jnp.float32),
            pltpu.VMEM((4, DT, 8, 129), jnp.float32),
            pltpu.SemaphoreType.DMA,
            pltpu.SemaphoreType.DMA,
            pltpu.SemaphoreType.DMA,
            pltpu.SemaphoreType.DMA,
            pltpu.SemaphoreType.DMA,
        ],
        compiler_params=pltpu.CompilerParams(
            use_tc_tiling_on_sc=False, needs_layout_passes=False),
    )
    def sc_kernel(idx4_hbm, ttab_hbm, ptab_hbm, out_hbm,
                  idx0, idx1, pos_v, rows0, rows1, stg_v,
                  sg0, sg1, so, si0, si1):
        idx_v = (idx0, idx1)
        rows_v = (rows0, rows1)
        sg = (sg0, sg1)
        si = (si0, si1)

        bt = lax.axis_index("s") * _NC + lax.axis_index("c")
        pltpu.sync_copy(ptab_hbm, pos_v)

        iota = lax.iota(jnp.int32, _LANES)
        di_vec = iota & 7
        dt_vecs = [(iota >> 3) + 2 * dj for dj in range(_D // _LANES)]

        def idx_copy(g, pb):
            return pltpu.make_async_copy(
                idx4_hbm.at[bt, pl.ds(g * 512, 512)], idx_v[pb], si[pb])

        def gather(p):
            return pltpu.make_async_copy(
                ttab_hbm.at[idx_v[p]], rows_v[p], sg[p])

        def out_copy(s):
            return pltpu.make_async_copy(
                stg_v.at[:, :, :, pl.ds(0, 128)],
                out_hbm.at[pl.ds(s, 4), :, bt], so)

        n_g = _SEQ // 4
        pltpu.sync_copy(idx4_hbm.at[bt, pl.ds(0, 512)], idx_v[0])
        gather(0).start()
        idx_copy(1, 1).start()

        def gg_body(gg, carry):
            for pg in range(2):
                g = gg * 2 + pg
                gather(pg).wait()
                @pl.when(g + 1 < n_g)
                def _():
                    idx_copy(g + 1, 1 - pg).wait()
                    gather(1 - pg).start()
                @pl.when(g + 2 < n_g)
                def _():
                    idx_copy(g + 2, pg).start()

                rows = rows_v[pg]
                s0 = g * 4
                @pl.when(g > 0)
                def _():
                    out_copy(s0).wait()
                for quarter in range(4):
                    stg = stg_v.at[quarter]
                    pvs = [pos_v[s0 + quarter, pl.ds(dj * _LANES, _LANES)]
                           for dj in range(_D // _LANES)]

                    @plsc.parallel_loop(0, 128, unroll=8)
                    def _(bi, _off=quarter * 128, _stg=stg, _pvs=pvs):
                        bi_vec = jnp.broadcast_to(bi, (_LANES,))
                        for dj in range(_D // _LANES):
                            v = rows[_off + bi,
                                     pl.ds(dj * _LANES, _LANES)]
                            plsc.store_scatter(
                                _stg, [dt_vecs[dj], di_vec, bi_vec],
                                v * _SCALE + _pvs[dj])
                out_copy(s0).start()
            return carry

        lax.fori_loop(0, n_g // 2, gg_body, 0, unroll=False)
        out_copy(_SEQ - 4).wait()

    return sc_kernel


@jax.jit
def kernel(inputs, token_table, position_table):
    batch, seq = inputs.shape
    v, d = token_table.shape
    st, bt = seq // 8, batch // 128
    idx4 = (inputs.T.reshape(st, 8, bt, 128).transpose(2, 0, 1, 3)
            .reshape(bt, seq * 128).astype(jnp.int32))
    fn = _build_sc_gather(batch, v)
    out5 = fn(idx4, token_table, position_table)
    out = out5.transpose(0, 1, 3, 2, 4).reshape(seq, d, batch)
    return out.transpose(2, 0, 1)


# speedup vs baseline: 1.0275x; 1.0275x over previous
import functools

import jax
import jax.numpy as jnp
from jax import lax
from jax.experimental import pallas as pl
from jax.experimental.pallas import tpu as pltpu
from jax.experimental.pallas import tpu_sc as plsc

_SEQ = 200
_D = 64
_LANES = 16
_SCALE = 8.0

_info = plsc.get_sparse_core_info()
_NC = _info.num_cores
_NS = _info.num_subcores
_NW = _NC * _NS


def _build_sc_gather(BATCH: int, V: int):
    ST = _SEQ // 8
    BT = BATCH // 128
    DT = _D // 8
    assert BT == _NW

    mesh = plsc.VectorSubcoreMesh(core_axis_name="c", subcore_axis_name="s")

    @functools.partial(
        pl.kernel,
        out_type=jax.ShapeDtypeStruct((_SEQ, DT, BT, 8, 128), jnp.float32),
        mesh=mesh,
        scratch_types=[
            pltpu.VMEM((512,), jnp.int32),
            pltpu.VMEM((512,), jnp.int32),
            pltpu.VMEM((_SEQ, _D), jnp.float32),
            pltpu.VMEM((512, _D), jnp.float32),
            pltpu.VMEM((512, _D), jnp.float32),
            pltpu.VMEM((2, DT, 8, 129), jnp.float32),
            pltpu.VMEM((2, DT, 8, 129), jnp.float32),
            pltpu.SemaphoreType.DMA,
            pltpu.SemaphoreType.DMA,
            pltpu.SemaphoreType.DMA,
            pltpu.SemaphoreType.DMA,
            pltpu.SemaphoreType.DMA,
            pltpu.SemaphoreType.DMA,
        ],
        compiler_params=pltpu.CompilerParams(
            use_tc_tiling_on_sc=False, needs_layout_passes=False),
    )
    def sc_kernel(idx4_hbm, ttab_hbm, ptab_hbm, out_hbm,
                  idx0, idx1, pos_v, rows0, rows1, stg0, stg1,
                  sg0, sg1, so0, so1, si0, si1):
        idx_v = (idx0, idx1)
        rows_v = (rows0, rows1)
        stg_v = (stg0, stg1)
        sg = (sg0, sg1)
        so = (so0, so1)
        si = (si0, si1)

        bt = lax.axis_index("s") * _NC + lax.axis_index("c")
        pltpu.sync_copy(ptab_hbm, pos_v)

        iota = lax.iota(jnp.int32, _LANES)
        di_vec = iota & 7
        dt_vecs = [(iota >> 3) + 2 * dj for dj in range(_D // _LANES)]

        def idx_copy(g, pb):
            return pltpu.make_async_copy(
                idx4_hbm.at[bt, pl.ds(g * 512, 512)], idx_v[pb], si[pb])

        def gather(p):
            return pltpu.make_async_copy(
                ttab_hbm.at[idx_v[p]], rows_v[p], sg[p])

        def out_copy(s, p):
            return pltpu.make_async_copy(
                stg_v[p].at[:, :, :, pl.ds(0, 128)],
                out_hbm.at[pl.ds(s, 2), :, bt], so[p])

        n_g = _SEQ // 4
        pltpu.sync_copy(idx4_hbm.at[bt, pl.ds(0, 512)], idx_v[0])
        gather(0).start()
        idx_copy(1, 1).start()

        def gg_body(gg, carry):
            for pg in range(2):
                g = gg * 2 + pg
                gather(pg).wait()
                @pl.when(g + 1 < n_g)
                def _():
                    idx_copy(g + 1, 1 - pg).wait()
                    gather(1 - pg).start()
                @pl.when(g + 2 < n_g)
                def _():
                    idx_copy(g + 2, pg).start()

                rows = rows_v[pg]
                for sub in range(2):
                    s = g * 4 + sub * 2
                    po = sub
                    @pl.when(g > 0)
                    def _():
                        out_copy(s, po).wait()
                    for half in range(2):
                        stg = stg_v[po].at[half]
                        pvs = [pos_v[s + half, pl.ds(dj * _LANES, _LANES)]
                               for dj in range(_D // _LANES)]

                        @plsc.parallel_loop(0, 128, unroll=8)
                        def _(bi, _off=sub * 256 + half * 128,
                              _stg=stg, _pvs=pvs):
                            bi_vec = jnp.broadcast_to(bi, (_LANES,))
                            for dj in range(_D // _LANES):
                                v = rows[_off + bi,
                                         pl.ds(dj * _LANES, _LANES)]
                                plsc.store_scatter(
                                    _stg, [dt_vecs[dj], di_vec, bi_vec],
                                    v * _SCALE + _pvs[dj])
                    out_copy(s, po).start()
            return carry

        lax.fori_loop(0, n_g // 2, gg_body, 0, unroll=False)
        out_copy(_SEQ - 4, 0).wait()
        out_copy(_SEQ - 2, 1).wait()

    return sc_kernel


@jax.jit
def kernel(inputs, token_table, position_table):
    batch, seq = inputs.shape
    v, d = token_table.shape
    st, bt = seq // 8, batch // 128
    idx4 = (inputs.T.reshape(st, 8, bt, 128).transpose(2, 0, 1, 3)
            .reshape(bt, seq * 128).astype(jnp.int32))
    fn = _build_sc_gather(batch, v)
    out5 = fn(idx4, token_table, position_table)
    out = out5.transpose(0, 1, 3, 2, 4).reshape(seq, d, batch)
    return out.transpose(2, 0, 1)
